# baseline (speedup 1.0000x reference)
import functools

import jax
import jax.numpy as jnp
from jax import lax
from jax.experimental import pallas as pl
from jax.experimental.pallas import tpu as pltpu
from jax.experimental.pallas import tpu_sc as plsc

N = 10000
E = 320000
NC = 2
NS = 16
NT = NC * NS
NPT = 313
NPAD = NT * NPT
ACC_ROWS = 320
DUMMY = 316
SENT = N
NSENT = 1024
XROWS = N + NSENT
CAP = 12800
CAP_S = 14336
HB = 320
ECHUNK = 3200
NECH = E // ECHUNK

_MESH = plsc.VectorSubcoreMesh(core_axis_name="c", subcore_axis_name="s")


def _wid():
    return lax.axis_index("s") * NC + lax.axis_index("c")


def _ranks(d, stage):
    iota = lax.iota(jnp.int32, 16)
    prev_idx = jnp.maximum(iota - 1, 0)
    ds, perm = plsc.sort_key_val(d, iota)
    stage[pl.ds(0, 16)] = ds
    prev = plsc.load_gather(stage.at[pl.ds(0, 16)], [prev_idx])
    newr = (iota == 0) | (ds != prev)
    rank = iota - plsc.cummax(jnp.where(newr, iota, 0))
    dsr = lax.rev(ds, (0,))
    stage[pl.ds(16, 16)] = dsr
    prevr = plsc.load_gather(stage.at[pl.ds(16, 16)], [prev_idx])
    newrr = (iota == 0) | (dsr != prevr)
    rankr = lax.rev(iota - plsc.cummax(jnp.where(newrr, iota, 0)), (0,))
    total = rank + rankr + 1
    return ds, perm, rank, total


def _bucket_body(ei_hbm, ssort_hbm, dsort_hbm, cnts_hbm,
                 src_v, dst_v, sbuf, dbuf, ssort, dsort,
                 hist, base, cnt2, stage, cnt_v, sem_a, sem_b):
    wid = _wid()
    lo = wid * NPT
    hi = lo + NPT
    zeros16 = jnp.zeros((16,), jnp.int32)
    dummy16 = jnp.full((16,), DUMMY, jnp.int32)
    sent16 = jnp.full((16,), SENT, jnp.int32)

    def init_a(i, c):
        sbuf[pl.ds(i * 16, 16)] = zeros16
        dbuf[pl.ds(i * 16, 16)] = dummy16
        return c
    lax.fori_loop(0, CAP // 16, init_a, 0)

    iota16 = lax.iota(jnp.int32, 16)

    def init_b(i, c):
        ssort[pl.ds(i * 16, 16)] = sent16 + ((iota16 + i * 16) & (NSENT - 1))
        dsort[pl.ds(i * 16, 16)] = dummy16
        return c
    lax.fori_loop(0, CAP_S // 16, init_b, 0)

    for i in range(HB // 16):
        hist[pl.ds(i * 16, 16)] = zeros16
        cnt2[pl.ds(i * 16, 16)] = zeros16

    def _fire(g, par, sem):
        pltpu.async_copy(ei_hbm.at[0, pl.ds(g * ECHUNK, ECHUNK)],
                         src_v.at[par], sem)
        pltpu.async_copy(ei_hbm.at[1, pl.ds(g * ECHUNK, ECHUNK)],
                         dst_v.at[par], sem)

    def _wait(par, sem):
        pltpu.make_async_copy(ei_hbm.at[0, pl.ds(0, ECHUNK)],
                              src_v.at[par], sem).wait()
        pltpu.make_async_copy(ei_hbm.at[1, pl.ds(0, ECHUNK)],
                              dst_v.at[par], sem).wait()

    _fire(0, 0, sem_a)

    def scan_pair(p, off):
        for par, sem, osem in ((0, sem_a, sem_b), (1, sem_b, sem_a)):
            g = 2 * p + par
            _wait(par, sem)
            nxt = g + 1

            @pl.when(nxt < NECH)
            def _():
                _fire(nxt, 1 - par, osem)

            def grp(j, off):
                d = dst_v[par, pl.ds(j * 16, 16)]
                m = (d >= lo) & (d < hi)
                nhit = plsc.all_reduce_population_count(m)[0]

                @pl.when(nhit > 0)
                def _():
                    s = src_v[par, pl.ds(j * 16, 16)]
                    offc = jnp.minimum(off, CAP - 16)
                    cs = jnp.cumsum(m.astype(jnp.int32))
                    pos = jnp.where(m, offc + cs - 1, CAP - 1)
                    plsc.store_scatter(sbuf, [pos], s)
                    plsc.store_scatter(dbuf, [pos], d - lo)
                return off + nhit
            off = lax.fori_loop(0, ECHUNK // 16, grp, off)
        return off

    off = lax.fori_loop(0, NECH // 2, scan_pair, jnp.int32(0))

    offc = jnp.minimum(off, CAP - 16)
    sbuf[pl.ds(offc, 16)] = zeros16
    dbuf[pl.ds(offc, 16)] = dummy16
    cnt_pad = jnp.minimum((off + 15) & (-16), CAP)

    def hist_body(g, c):
        d = dbuf[pl.ds(g * 16, 16)]
        ds, _, _, total = _ranks(d, stage)
        hcur = plsc.load_gather(hist, [ds])
        plsc.store_scatter(hist, [ds], hcur + total)
        return c
    lax.fori_loop(0, cnt_pad // 16, hist_body, 0)

    carry = jnp.int32(0)
    for i in range(HB // 16):
        h = hist[pl.ds(i * 16, 16)]
        hp = (h + 15) & (-16)
        s = jnp.cumsum(hp)
        base[pl.ds(i * 16, 16)] = carry + s - hp
        carry = carry + s[15]
    total_pad = jnp.minimum(carry, CAP_S)

    def place_body(g, c):
        d = dbuf[pl.ds(g * 16, 16)]
        s = sbuf[pl.ds(g * 16, 16)]
        ds, perm, rank, total = _ranks(d, stage)
        stage[pl.ds(0, 16)] = s
        ssv = plsc.load_gather(stage.at[pl.ds(0, 16)], [perm])
        b = plsc.load_gather(base, [ds])
        cprev = plsc.load_gather(cnt2, [ds])
        pos = jnp.minimum(b + cprev + rank, CAP_S - 1)
        plsc.store_scatter(ssort, [pos], ssv)
        plsc.store_scatter(dsort, [pos], ds)
        plsc.store_scatter(cnt2, [ds], cprev + total)
        return c
    lax.fori_loop(0, cnt_pad // 16, place_body, 0)

    cnt_v[...] = jnp.zeros((16,), jnp.int32) + total_pad
    pltpu.sync_copy(ssort, ssort_hbm.at[wid])
    pltpu.sync_copy(dsort, dsort_hbm.at[wid])
    pltpu.sync_copy(cnt_v, cnts_hbm.at[wid])


_bucket = pl.kernel(
    _bucket_body,
    out_type=[jax.ShapeDtypeStruct((NT, CAP_S), jnp.int32),
              jax.ShapeDtypeStruct((NT, CAP_S), jnp.int32),
              jax.ShapeDtypeStruct((NT, 16), jnp.int32)],
    mesh=_MESH,
    scratch_types=[pltpu.VMEM((2, ECHUNK), jnp.int32),
                   pltpu.VMEM((2, ECHUNK), jnp.int32),
                   pltpu.VMEM((CAP,), jnp.int32),
                   pltpu.VMEM((CAP,), jnp.int32),
                   pltpu.VMEM((CAP_S,), jnp.int32),
                   pltpu.VMEM((CAP_S,), jnp.int32),
                   pltpu.VMEM((HB,), jnp.int32),
                   pltpu.VMEM((HB,), jnp.int32),
                   pltpu.VMEM((HB,), jnp.int32),
                   pltpu.VMEM((32,), jnp.int32),
                   pltpu.VMEM((16,), jnp.int32),
                   pltpu.SemaphoreType.DMA,
                   pltpu.SemaphoreType.DMA],
    compiler_params=pltpu.CompilerParams(needs_layout_passes=False),
    name="edge_bucket",
)


def _segmax_body(C, GC, x_hbm, ssort_hbm, dsort_hbm, cnts_hbm, out_hbm,
                 src_v, dgrp_v, rows_v, acc, cnt_v, sem_a, sem_b):
    NREG = C // 16
    wid = _wid()
    pltpu.sync_copy(cnts_hbm.at[wid], cnt_v)
    cnt = cnt_v[pl.ds(0, 16)][0]

    ninf = jnp.full((16,), -jnp.inf, jnp.float32)

    def init_body(i, c):
        acc[pl.ds(i * 16, 16)] = ninf
        return c
    lax.fori_loop(0, ACC_ROWS * C // 16, init_body, 0)

    nchunks = (cnt + GC - 1) // GC
    sems = (sem_a, sem_b)

    def _lin(g, par):
        pltpu.sync_copy(ssort_hbm.at[wid, pl.ds(g * GC, GC)], src_v.at[par])
        pltpu.sync_copy(dsort_hbm.at[wid, pl.ds(g * GC, GC)], dgrp_v.at[par])

    def _fire(par, sem):
        for j in range(GC // 128):
            pltpu.async_copy(
                x_hbm.at[src_v.at[par].at[pl.ds(j * 128, 128)]],
                rows_v.at[pl.ds(par * GC + j * 128, 128)], sem)

    def _wait(par, sem):
        for j in range(GC // 128):
            pltpu.make_async_copy(
                x_hbm.at[src_v.at[par].at[pl.ds(j * 128, 128)]],
                rows_v.at[pl.ds(par * GC + j * 128, 128)], sem).wait()

    _lin(0, 0)
    _fire(0, sem_a)

    def pair(p, c):
        for par in (0, 1):
            g = 2 * p + par

            @pl.when(g < nchunks)
            def _():
                nxt = g + 1

                @pl.when(nxt < nchunks)
                def _():
                    _lin(nxt, 1 - par)
                    _fire(1 - par, sems[1 - par])

                _wait(par, sems[par])
                ng = jnp.minimum(GC, cnt - g * GC) // 16

                def grp(i, c2):
                    nd = dgrp_v[par, pl.ds(i * 16, 16)][0]
                    rowbase = nd * C
                    for k in range(NREG):
                        vs = [rows_v[par * GC + i * 16 + j, pl.ds(k * 16, 16)]
                              for j in range(16)]
                        while len(vs) > 1:
                            vs = [jnp.maximum(vs[2 * a], vs[2 * a + 1])
                                  for a in range(len(vs) // 2)]
                        a = acc[pl.ds(rowbase + k * 16, 16)]
                        acc[pl.ds(rowbase + k * 16, 16)] = jnp.maximum(a, vs[0])
                    return c2
                lax.fori_loop(0, ng, grp, 0)
        return c

    lax.fori_loop(0, (nchunks + 1) // 2, pair, 0)

    NLAST = N - (NT - 1) * NPT

    @pl.when(wid < NT - 1)
    def _():
        pltpu.sync_copy(acc.at[pl.ds(0, NPT * C)],
                        out_hbm.at[pl.ds(wid * NPT * C, NPT * C)])

    @pl.when(wid == NT - 1)
    def _():
        pltpu.sync_copy(acc.at[pl.ds(0, NLAST * C)],
                        out_hbm.at[pl.ds((NT - 1) * NPT * C, NLAST * C)])


def _make_segmax(C, GC):
    return pl.kernel(
        functools.partial(_segmax_body, C, GC),
        out_type=jax.ShapeDtypeStruct((N * C,), jnp.float32),
        mesh=_MESH,
        scratch_types=[pltpu.VMEM((2, GC), jnp.int32),
                       pltpu.VMEM((2, GC), jnp.int32),
                       pltpu.VMEM((2 * GC, C), jnp.float32),
                       pltpu.VMEM((ACC_ROWS * C,), jnp.float32),
                       pltpu.VMEM((16,), jnp.int32),
                       pltpu.SemaphoreType.DMA,
                       pltpu.SemaphoreType.DMA],
        compiler_params=pltpu.CompilerParams(needs_layout_passes=False,
                                             use_tc_tiling_on_sc=False),
        name=f"segmax{C}",
    )


_segmax128 = _make_segmax(128, 256)
_segmax64 = _make_segmax(64, 512)


def _layer_call(C, residual, x, m, W1, W2, b):
    BN = 1000

    def body(x_ref, m_ref, w1_ref, w2_ref, b_ref, o_ref):
        xb = x_ref[...]
        mb = m_ref[...]
        agg = jnp.where(mb == -jnp.inf, 0.0, mb - xb)
        h = (jnp.dot(xb, w1_ref[...], preferred_element_type=jnp.float32)
             + jnp.dot(agg, w2_ref[...], preferred_element_type=jnp.float32)
             + b_ref[...])
        h = jnp.maximum(h, 0.0)
        if residual:
            h = h + xb
        o_ref[...] = h

    return pl.pallas_call(
        body,
        grid=(N // BN,),
        in_specs=[pl.BlockSpec((BN, C), lambda i: (i, 0)),
                  pl.BlockSpec((BN, C), lambda i: (i, 0)),
                  pl.BlockSpec((C, 64), lambda i: (0, 0)),
                  pl.BlockSpec((C, 64), lambda i: (0, 0)),
                  pl.BlockSpec((1, 64), lambda i: (0, 0))],
        out_specs=pl.BlockSpec((BN, 64), lambda i: (i, 0)),
        out_shape=jax.ShapeDtypeStruct((N, 64), jnp.float32),
    )(x, m, W1, W2, b)


def _tail_call(feats, fusion_W, fusion_b, p1f, p1v, p1b, W2, b2, W3, b3):
    BN = 400

    def body(f_ref, fw_ref, fb_ref, p1f_ref, p1v_ref, p1b_ref,
             w2_ref, b2_ref, w3_ref, b3_ref, o_ref):
        fb = f_ref[...]
        t = jnp.maximum(
            jnp.dot(fb, fw_ref[...], preferred_element_type=jnp.float32)
            + fb_ref[...], 0.0)
        fu = jnp.max(t, axis=1, keepdims=True)
        h1 = jnp.maximum(
            jnp.dot(fb, p1f_ref[...], preferred_element_type=jnp.float32)
            + fu * p1v_ref[...] + p1b_ref[...], 0.0)
        h2 = jnp.maximum(
            jnp.dot(h1, w2_ref[...], preferred_element_type=jnp.float32)
            + b2_ref[...], 0.0)
        o_ref[...] = (jnp.dot(h2, w3_ref[...], preferred_element_type=jnp.float32)
                      + b3_ref[...])

    F = feats.shape[1]
    return pl.pallas_call(
        body,
        grid=(N // BN,),
        in_specs=[pl.BlockSpec((BN, F), lambda i: (i, 0)),
                  pl.BlockSpec((F, 1024), lambda i: (0, 0)),
                  pl.BlockSpec((1, 1024), lambda i: (0, 0)),
                  pl.BlockSpec((F, 512), lambda i: (0, 0)),
                  pl.BlockSpec((1, 512), lambda i: (0, 0)),
                  pl.BlockSpec((1, 512), lambda i: (0, 0)),
                  pl.BlockSpec((512, 256), lambda i: (0, 0)),
                  pl.BlockSpec((1, 256), lambda i: (0, 0)),
                  pl.BlockSpec((256, 13), lambda i: (0, 0)),
                  pl.BlockSpec((1, 13), lambda i: (0, 0))],
        out_specs=pl.BlockSpec((BN, 13), lambda i: (i, 0)),
        out_shape=jax.ShapeDtypeStruct((N, 13), jnp.float32),
    )(feats, fusion_W, fusion_b, p1f, p1v, p1b, W2, b2, W3, b3)


def _pad_ninf(x):
    return jnp.concatenate(
        [x, jnp.full((XROWS - N, x.shape[1]), -jnp.inf, x.dtype)], axis=0)


def kernel(x, edge_index, batch, head_W, head_b, blocks_W, blocks_b,
           fusion_W, fusion_b, pred1_W, pred1_b, pred2_W, pred2_b,
           pred3_W, pred3_b):
    ssort, dsort, cnts = _bucket(edge_index)

    m0 = _segmax128(_pad_ninf(x), ssort, dsort, cnts).reshape(N, 128)
    h = _layer_call(128, False, x, m0, head_W[:128], head_W[128:],
                    head_b.reshape(1, 64))
    feats = [h]
    for i in range(6):
        m = _segmax64(_pad_ninf(h), ssort, dsort, cnts).reshape(N, 64)
        h = _layer_call(64, True, h, m, blocks_W[i, :64], blocks_W[i, 64:],
                        blocks_b[i].reshape(1, 64))
        feats.append(h)
    feats = jnp.concatenate(feats, axis=1)

    return _tail_call(feats, fusion_W, fusion_b.reshape(1, 1024),
                      pred1_W[:448], pred1_W[448:449], pred1_b.reshape(1, 512),
                      pred2_W, pred2_b.reshape(1, 256),
                      pred3_W, pred3_b.reshape(1, 13))


# speedup vs baseline: 1.1797x; 1.1797x over previous
import functools

import jax
import jax.numpy as jnp
from jax import lax
from jax.experimental import pallas as pl
from jax.experimental.pallas import tpu as pltpu
from jax.experimental.pallas import tpu_sc as plsc

N = 10000
E = 320000
NC = 2
NS = 16
NT = NC * NS
NPT = 313
NPAD = NT * NPT
ACC_ROWS = 320
DUMMY = 316
SENT = N
NSENT = 1024
XROWS = N + NSENT
CAP = 12800
CAP_S = 14336
HB = 320
ECHUNK = 3200
NECH = E // ECHUNK

_MESH = plsc.VectorSubcoreMesh(core_axis_name="c", subcore_axis_name="s")


def _wid():
    return lax.axis_index("s") * NC + lax.axis_index("c")


def _ranks(d, stage):
    iota = lax.iota(jnp.int32, 16)
    prev_idx = jnp.maximum(iota - 1, 0)
    ds, perm = plsc.sort_key_val(d, iota)
    stage[pl.ds(0, 16)] = ds
    prev = plsc.load_gather(stage.at[pl.ds(0, 16)], [prev_idx])
    newr = (iota == 0) | (ds != prev)
    rank = iota - plsc.cummax(jnp.where(newr, iota, 0))
    dsr = lax.rev(ds, (0,))
    stage[pl.ds(16, 16)] = dsr
    prevr = plsc.load_gather(stage.at[pl.ds(16, 16)], [prev_idx])
    newrr = (iota == 0) | (dsr != prevr)
    rankr = lax.rev(iota - plsc.cummax(jnp.where(newrr, iota, 0)), (0,))
    total = rank + rankr + 1
    return ds, perm, rank, total


def _bucket_body(ei_hbm, ssort_hbm, dsort_hbm, cnts_hbm,
                 src_v, dst_v, sbuf, dbuf, ssort, dsort,
                 hist, base, cnt2, stage, cnt_v, sem_a, sem_b):
    wid = _wid()
    lo = wid * NPT
    hi = lo + NPT
    zeros16 = jnp.zeros((16,), jnp.int32)
    dummy16 = jnp.full((16,), DUMMY, jnp.int32)
    sent16 = jnp.full((16,), SENT, jnp.int32)

    def init_a(i, c):
        sbuf[pl.ds(i * 16, 16)] = zeros16
        dbuf[pl.ds(i * 16, 16)] = dummy16
        return c
    lax.fori_loop(0, CAP // 16, init_a, 0)

    iota16 = lax.iota(jnp.int32, 16)

    def init_b(i, c):
        ssort[pl.ds(i * 16, 16)] = sent16 + ((iota16 + i * 16) & (NSENT - 1))
        dsort[pl.ds(i * 16, 16)] = dummy16
        return c
    lax.fori_loop(0, CAP_S // 16, init_b, 0)

    for i in range(HB // 16):
        hist[pl.ds(i * 16, 16)] = zeros16
        cnt2[pl.ds(i * 16, 16)] = zeros16

    def _fire(g, par, sem):
        pltpu.async_copy(ei_hbm.at[0, pl.ds(g * ECHUNK, ECHUNK)],
                         src_v.at[par], sem)
        pltpu.async_copy(ei_hbm.at[1, pl.ds(g * ECHUNK, ECHUNK)],
                         dst_v.at[par], sem)

    def _wait(par, sem):
        pltpu.make_async_copy(ei_hbm.at[0, pl.ds(0, ECHUNK)],
                              src_v.at[par], sem).wait()
        pltpu.make_async_copy(ei_hbm.at[1, pl.ds(0, ECHUNK)],
                              dst_v.at[par], sem).wait()

    _fire(0, 0, sem_a)

    def scan_pair(p, off):
        for par, sem, osem in ((0, sem_a, sem_b), (1, sem_b, sem_a)):
            g = 2 * p + par
            _wait(par, sem)
            nxt = g + 1

            @pl.when(nxt < NECH)
            def _():
                _fire(nxt, 1 - par, osem)

            def grp(j, off):
                d = dst_v[par, pl.ds(j * 16, 16)]
                s = src_v[par, pl.ds(j * 16, 16)]
                m = (d >= lo) & (d < hi)
                offc = jnp.minimum(off, CAP - 16)
                cs = jnp.cumsum(m.astype(jnp.int32))
                pos = jnp.where(m, offc + cs - 1, CAP - 1)
                plsc.store_scatter(sbuf, [pos], s)
                plsc.store_scatter(dbuf, [pos], d - lo)
                return off + cs[15]
            off = lax.fori_loop(0, ECHUNK // 16, grp, off)
        return off

    off = lax.fori_loop(0, NECH // 2, scan_pair, jnp.int32(0))

    offc = jnp.minimum(off, CAP - 16)
    sbuf[pl.ds(offc, 16)] = zeros16
    dbuf[pl.ds(offc, 16)] = dummy16
    cnt_pad = jnp.minimum((off + 15) & (-16), CAP)

    def hist_body(g, c):
        d = dbuf[pl.ds(g * 16, 16)]
        ds, _, _, total = _ranks(d, stage)
        hcur = plsc.load_gather(hist, [ds])
        plsc.store_scatter(hist, [ds], hcur + total)
        return c
    lax.fori_loop(0, cnt_pad // 16, hist_body, 0)

    carry = jnp.int32(0)
    for i in range(HB // 16):
        h = hist[pl.ds(i * 16, 16)]
        hp = (h + 15) & (-16)
        s = jnp.cumsum(hp)
        base[pl.ds(i * 16, 16)] = carry + s - hp
        carry = carry + s[15]
    total_pad = jnp.minimum(carry, CAP_S)

    def place_body(g, c):
        d = dbuf[pl.ds(g * 16, 16)]
        s = sbuf[pl.ds(g * 16, 16)]
        ds, perm, rank, total = _ranks(d, stage)
        stage[pl.ds(0, 16)] = s
        ssv = plsc.load_gather(stage.at[pl.ds(0, 16)], [perm])
        b = plsc.load_gather(base, [ds])
        cprev = plsc.load_gather(cnt2, [ds])
        pos = jnp.minimum(b + cprev + rank, CAP_S - 1)
        plsc.store_scatter(ssort, [pos], ssv)
        plsc.store_scatter(dsort, [pos], ds)
        plsc.store_scatter(cnt2, [ds], cprev + total)
        return c
    lax.fori_loop(0, cnt_pad // 16, place_body, 0)

    cnt_v[...] = jnp.zeros((16,), jnp.int32) + total_pad
    pltpu.sync_copy(ssort, ssort_hbm.at[wid])
    pltpu.sync_copy(dsort, dsort_hbm.at[wid])
    pltpu.sync_copy(cnt_v, cnts_hbm.at[wid])


_bucket = pl.kernel(
    _bucket_body,
    out_type=[jax.ShapeDtypeStruct((NT, CAP_S), jnp.int32),
              jax.ShapeDtypeStruct((NT, CAP_S), jnp.int32),
              jax.ShapeDtypeStruct((NT, 16), jnp.int32)],
    mesh=_MESH,
    scratch_types=[pltpu.VMEM((2, ECHUNK), jnp.int32),
                   pltpu.VMEM((2, ECHUNK), jnp.int32),
                   pltpu.VMEM((CAP,), jnp.int32),
                   pltpu.VMEM((CAP,), jnp.int32),
                   pltpu.VMEM((CAP_S,), jnp.int32),
                   pltpu.VMEM((CAP_S,), jnp.int32),
                   pltpu.VMEM((HB,), jnp.int32),
                   pltpu.VMEM((HB,), jnp.int32),
                   pltpu.VMEM((HB,), jnp.int32),
                   pltpu.VMEM((32,), jnp.int32),
                   pltpu.VMEM((16,), jnp.int32),
                   pltpu.SemaphoreType.DMA,
                   pltpu.SemaphoreType.DMA],
    compiler_params=pltpu.CompilerParams(needs_layout_passes=False),
    name="edge_bucket",
)


def _segmax_body(C, GC, x_hbm, ssort_hbm, dsort_hbm, cnts_hbm, out_hbm,
                 src_v, dgrp_v, rows_v, acc, cnt_v, sem_a, sem_b):
    NREG = C // 16
    wid = _wid()
    pltpu.sync_copy(cnts_hbm.at[wid], cnt_v)
    cnt = cnt_v[pl.ds(0, 16)][0]

    ninf = jnp.full((16,), -jnp.inf, jnp.float32)

    def init_body(i, c):
        acc[pl.ds(i * 16, 16)] = ninf
        return c
    lax.fori_loop(0, ACC_ROWS * C // 16, init_body, 0)

    nchunks = (cnt + GC - 1) // GC
    sems = (sem_a, sem_b)

    def _lin(g, par):
        pltpu.sync_copy(ssort_hbm.at[wid, pl.ds(g * GC, GC)], src_v.at[par])
        pltpu.sync_copy(dsort_hbm.at[wid, pl.ds(g * GC, GC)], dgrp_v.at[par])

    def _fire(par, sem):
        for j in range(GC // 128):
            pltpu.async_copy(
                x_hbm.at[src_v.at[par].at[pl.ds(j * 128, 128)]],
                rows_v.at[pl.ds(par * GC + j * 128, 128)], sem)

    def _wait(par, sem):
        for j in range(GC // 128):
            pltpu.make_async_copy(
                x_hbm.at[src_v.at[par].at[pl.ds(j * 128, 128)]],
                rows_v.at[pl.ds(par * GC + j * 128, 128)], sem).wait()

    _lin(0, 0)
    _fire(0, sem_a)

    def pair(p, c):
        for par in (0, 1):
            g = 2 * p + par

            @pl.when(g < nchunks)
            def _():
                nxt = g + 1

                @pl.when(nxt < nchunks)
                def _():
                    _lin(nxt, 1 - par)
                    _fire(1 - par, sems[1 - par])

                _wait(par, sems[par])
                ng = jnp.minimum(GC, cnt - g * GC) // 16

                def grp(i, c2):
                    nd = dgrp_v[par, pl.ds(i * 16, 16)][0]
                    rowbase = nd * C
                    for k in range(NREG):
                        vs = [rows_v[par * GC + i * 16 + j, pl.ds(k * 16, 16)]
                              for j in range(16)]
                        while len(vs) > 1:
                            vs = [jnp.maximum(vs[2 * a], vs[2 * a + 1])
                                  for a in range(len(vs) // 2)]
                        a = acc[pl.ds(rowbase + k * 16, 16)]
                        acc[pl.ds(rowbase + k * 16, 16)] = jnp.maximum(a, vs[0])
                    return c2
                lax.fori_loop(0, ng, grp, 0)
        return c

    lax.fori_loop(0, (nchunks + 1) // 2, pair, 0)

    NLAST = N - (NT - 1) * NPT

    @pl.when(wid < NT - 1)
    def _():
        pltpu.sync_copy(acc.at[pl.ds(0, NPT * C)],
                        out_hbm.at[pl.ds(wid * NPT * C, NPT * C)])

    @pl.when(wid == NT - 1)
    def _():
        pltpu.sync_copy(acc.at[pl.ds(0, NLAST * C)],
                        out_hbm.at[pl.ds((NT - 1) * NPT * C, NLAST * C)])


def _make_segmax(C, GC):
    return pl.kernel(
        functools.partial(_segmax_body, C, GC),
        out_type=jax.ShapeDtypeStruct((N * C,), jnp.float32),
        mesh=_MESH,
        scratch_types=[pltpu.VMEM((2, GC), jnp.int32),
                       pltpu.VMEM((2, GC), jnp.int32),
                       pltpu.VMEM((2 * GC, C), jnp.float32),
                       pltpu.VMEM((ACC_ROWS * C,), jnp.float32),
                       pltpu.VMEM((16,), jnp.int32),
                       pltpu.SemaphoreType.DMA,
                       pltpu.SemaphoreType.DMA],
        compiler_params=pltpu.CompilerParams(needs_layout_passes=False,
                                             use_tc_tiling_on_sc=False),
        name=f"segmax{C}",
    )


_segmax128 = _make_segmax(128, 256)
_segmax64 = _make_segmax(64, 512)


def _layer_call(C, residual, x, m, W1, W2, b):
    BN = 1000

    def body(x_ref, m_ref, w1_ref, w2_ref, b_ref, o_ref):
        xb = x_ref[...]
        mb = m_ref[...]
        agg = jnp.where(mb == -jnp.inf, 0.0, mb - xb)
        h = (jnp.dot(xb, w1_ref[...], preferred_element_type=jnp.float32)
             + jnp.dot(agg, w2_ref[...], preferred_element_type=jnp.float32)
             + b_ref[...])
        h = jnp.maximum(h, 0.0)
        if residual:
            h = h + xb
        o_ref[...] = h

    return pl.pallas_call(
        body,
        grid=(N // BN,),
        in_specs=[pl.BlockSpec((BN, C), lambda i: (i, 0)),
                  pl.BlockSpec((BN, C), lambda i: (i, 0)),
                  pl.BlockSpec((C, 64), lambda i: (0, 0)),
                  pl.BlockSpec((C, 64), lambda i: (0, 0)),
                  pl.BlockSpec((1, 64), lambda i: (0, 0))],
        out_specs=pl.BlockSpec((BN, 64), lambda i: (i, 0)),
        out_shape=jax.ShapeDtypeStruct((N, 64), jnp.float32),
    )(x, m, W1, W2, b)


def _tail_call(feats, fusion_W, fusion_b, p1f, p1v, p1b, W2, b2, W3, b3):
    BN = 400

    def body(f_ref, fw_ref, fb_ref, p1f_ref, p1v_ref, p1b_ref,
             w2_ref, b2_ref, w3_ref, b3_ref, o_ref):
        fb = f_ref[...]
        t = jnp.maximum(
            jnp.dot(fb, fw_ref[...], preferred_element_type=jnp.float32)
            + fb_ref[...], 0.0)
        fu = jnp.max(t, axis=1, keepdims=True)
        h1 = jnp.maximum(
            jnp.dot(fb, p1f_ref[...], preferred_element_type=jnp.float32)
            + fu * p1v_ref[...] + p1b_ref[...], 0.0)
        h2 = jnp.maximum(
            jnp.dot(h1, w2_ref[...], preferred_element_type=jnp.float32)
            + b2_ref[...], 0.0)
        o_ref[...] = (jnp.dot(h2, w3_ref[...], preferred_element_type=jnp.float32)
                      + b3_ref[...])

    F = feats.shape[1]
    return pl.pallas_call(
        body,
        grid=(N // BN,),
        in_specs=[pl.BlockSpec((BN, F), lambda i: (i, 0)),
                  pl.BlockSpec((F, 1024), lambda i: (0, 0)),
                  pl.BlockSpec((1, 1024), lambda i: (0, 0)),
                  pl.BlockSpec((F, 512), lambda i: (0, 0)),
                  pl.BlockSpec((1, 512), lambda i: (0, 0)),
                  pl.BlockSpec((1, 512), lambda i: (0, 0)),
                  pl.BlockSpec((512, 256), lambda i: (0, 0)),
                  pl.BlockSpec((1, 256), lambda i: (0, 0)),
                  pl.BlockSpec((256, 13), lambda i: (0, 0)),
                  pl.BlockSpec((1, 13), lambda i: (0, 0))],
        out_specs=pl.BlockSpec((BN, 13), lambda i: (i, 0)),
        out_shape=jax.ShapeDtypeStruct((N, 13), jnp.float32),
    )(feats, fusion_W, fusion_b, p1f, p1v, p1b, W2, b2, W3, b3)


def _pad_ninf(x):
    return jnp.concatenate(
        [x, jnp.full((XROWS - N, x.shape[1]), -jnp.inf, x.dtype)], axis=0)


def kernel(x, edge_index, batch, head_W, head_b, blocks_W, blocks_b,
           fusion_W, fusion_b, pred1_W, pred1_b, pred2_W, pred2_b,
           pred3_W, pred3_b):
    ssort, dsort, cnts = _bucket(edge_index)

    m0 = _segmax128(_pad_ninf(x), ssort, dsort, cnts).reshape(N, 128)
    h = _layer_call(128, False, x, m0, head_W[:128], head_W[128:],
                    head_b.reshape(1, 64))
    feats = [h]
    for i in range(6):
        m = _segmax64(_pad_ninf(h), ssort, dsort, cnts).reshape(N, 64)
        h = _layer_call(64, True, h, m, blocks_W[i, :64], blocks_W[i, 64:],
                        blocks_b[i].reshape(1, 64))
        feats.append(h)
    feats = jnp.concatenate(feats, axis=1)

    return _tail_call(feats, fusion_W, fusion_b.reshape(1, 1024),
                      pred1_W[:448], pred1_W[448:449], pred1_b.reshape(1, 512),
                      pred2_W, pred2_b.reshape(1, 256),
                      pred3_W, pred3_b.reshape(1, 13))


# speedup vs baseline: 1.2145x; 1.0295x over previous
import functools

import jax
import jax.numpy as jnp
from jax import lax
from jax.experimental import pallas as pl
from jax.experimental.pallas import tpu as pltpu
from jax.experimental.pallas import tpu_sc as plsc

N = 10000
E = 320000
NC = 2
NS = 16
NT = NC * NS
NPT = 313
NPAD = NT * NPT
ACC_ROWS = 320
DUMMY = 316
SENT = N
NSENT = 1024
XROWS = N + NSENT
CAP = 12800
CAP_S = 14592
HB = 320
ECHUNK = 6400
NECH = E // ECHUNK

_MESH = plsc.VectorSubcoreMesh(core_axis_name="c", subcore_axis_name="s")


def _wid():
    return lax.axis_index("s") * NC + lax.axis_index("c")


def _ranks(d, stage):
    iota = lax.iota(jnp.int32, 16)
    prev_idx = jnp.maximum(iota - 1, 0)
    ds, perm = plsc.sort_key_val(d, iota)
    stage[pl.ds(0, 16)] = ds
    prev = plsc.load_gather(stage.at[pl.ds(0, 16)], [prev_idx])
    newr = (iota == 0) | (ds != prev)
    rank = iota - plsc.cummax(jnp.where(newr, iota, 0))
    dsr = lax.rev(ds, (0,))
    stage[pl.ds(16, 16)] = dsr
    prevr = plsc.load_gather(stage.at[pl.ds(16, 16)], [prev_idx])
    newrr = (iota == 0) | (dsr != prevr)
    rankr = lax.rev(iota - plsc.cummax(jnp.where(newrr, iota, 0)), (0,))
    total = rank + rankr + 1
    return ds, perm, rank, total


def _bucket_body(ei_hbm, ssort_hbm, dsort_hbm, cnts_hbm,
                 src_v, dst_v, sbuf, dbuf, ssort, dsort,
                 hist, base, cnt2, stage, cnt_v, sem_a, sem_b):
    wid = _wid()
    lo = wid * NPT
    hi = lo + NPT
    zeros16 = jnp.zeros((16,), jnp.int32)
    dummy16 = jnp.full((16,), DUMMY, jnp.int32)
    sent16 = jnp.full((16,), SENT, jnp.int32)

    def init_a(i, c):
        sbuf[pl.ds(i * 16, 16)] = zeros16
        dbuf[pl.ds(i * 16, 16)] = dummy16
        return c
    lax.fori_loop(0, CAP // 16, init_a, 0)

    iota16 = lax.iota(jnp.int32, 16)

    def init_b(i, c):
        ssort[pl.ds(i * 16, 16)] = sent16 + ((iota16 + i * 16) & (NSENT - 1))
        dsort[pl.ds(i * 16, 16)] = dummy16
        return c
    lax.fori_loop(0, CAP_S // 16, init_b, 0)

    for i in range(HB // 16):
        hist[pl.ds(i * 16, 16)] = zeros16
        cnt2[pl.ds(i * 16, 16)] = zeros16

    def _fire(g, par, sem):
        pltpu.async_copy(ei_hbm.at[0, pl.ds(g * ECHUNK, ECHUNK)],
                         src_v.at[par], sem)
        pltpu.async_copy(ei_hbm.at[1, pl.ds(g * ECHUNK, ECHUNK)],
                         dst_v.at[par], sem)

    def _wait(par, sem):
        pltpu.make_async_copy(ei_hbm.at[0, pl.ds(0, ECHUNK)],
                              src_v.at[par], sem).wait()
        pltpu.make_async_copy(ei_hbm.at[1, pl.ds(0, ECHUNK)],
                              dst_v.at[par], sem).wait()

    _fire(0, 0, sem_a)

    def scan_pair(p, off):
        for par, sem, osem in ((0, sem_a, sem_b), (1, sem_b, sem_a)):
            g = 2 * p + par
            _wait(par, sem)
            nxt = g + 1

            @pl.when(nxt < NECH)
            def _():
                _fire(nxt, 1 - par, osem)

            def grp(j, off):
                d = dst_v[par, pl.ds(j * 16, 16)]
                s = src_v[par, pl.ds(j * 16, 16)]
                m = (d >= lo) & (d < hi)
                offc = jnp.minimum(off, CAP - 16)
                cs = jnp.cumsum(m.astype(jnp.int32))
                pos = jnp.where(m, offc + cs - 1, CAP - 1)
                plsc.store_scatter(sbuf, [pos], s)
                plsc.store_scatter(dbuf, [pos], d - lo)
                return off + cs[15]
            off = lax.fori_loop(0, ECHUNK // 16, grp, off)
        return off

    off = lax.fori_loop(0, NECH // 2, scan_pair, jnp.int32(0))

    offc = jnp.minimum(off, CAP - 16)
    sbuf[pl.ds(offc, 16)] = zeros16
    dbuf[pl.ds(offc, 16)] = dummy16
    cnt_pad = jnp.minimum((off + 15) & (-16), CAP)

    def hist_body(g, c):
        d = dbuf[pl.ds(g * 16, 16)]
        ds, _, _, total = _ranks(d, stage)
        hcur = plsc.load_gather(hist, [ds])
        plsc.store_scatter(hist, [ds], hcur + total)
        return c
    lax.fori_loop(0, cnt_pad // 16, hist_body, 0)

    carry = jnp.int32(0)
    for i in range(HB // 16):
        h = hist[pl.ds(i * 16, 16)]
        hp = (h + 15) & (-16)
        s = jnp.cumsum(hp)
        base[pl.ds(i * 16, 16)] = carry + s - hp
        carry = carry + s[15]
    total_pad = jnp.minimum(carry, CAP_S)

    def place_body(g, c):
        d = dbuf[pl.ds(g * 16, 16)]
        s = sbuf[pl.ds(g * 16, 16)]
        ds, perm, rank, total = _ranks(d, stage)
        stage[pl.ds(0, 16)] = s
        ssv = plsc.load_gather(stage.at[pl.ds(0, 16)], [perm])
        b = plsc.load_gather(base, [ds])
        cprev = plsc.load_gather(cnt2, [ds])
        pos = jnp.minimum(b + cprev + rank, CAP_S - 1)
        plsc.store_scatter(ssort, [pos], ssv)
        plsc.store_scatter(dsort, [pos], ds)
        plsc.store_scatter(cnt2, [ds], cprev + total)
        return c
    lax.fori_loop(0, cnt_pad // 16, place_body, 0)

    cnt_v[...] = jnp.zeros((16,), jnp.int32) + total_pad
    pltpu.sync_copy(ssort, ssort_hbm.at[wid])
    pltpu.sync_copy(dsort, dsort_hbm.at[wid])
    pltpu.sync_copy(cnt_v, cnts_hbm.at[wid])


_bucket = pl.kernel(
    _bucket_body,
    out_type=[jax.ShapeDtypeStruct((NT, CAP_S), jnp.int32),
              jax.ShapeDtypeStruct((NT, CAP_S), jnp.int32),
              jax.ShapeDtypeStruct((NT, 16), jnp.int32)],
    mesh=_MESH,
    scratch_types=[pltpu.VMEM((2, ECHUNK), jnp.int32),
                   pltpu.VMEM((2, ECHUNK), jnp.int32),
                   pltpu.VMEM((CAP,), jnp.int32),
                   pltpu.VMEM((CAP,), jnp.int32),
                   pltpu.VMEM((CAP_S,), jnp.int32),
                   pltpu.VMEM((CAP_S,), jnp.int32),
                   pltpu.VMEM((HB,), jnp.int32),
                   pltpu.VMEM((HB,), jnp.int32),
                   pltpu.VMEM((HB,), jnp.int32),
                   pltpu.VMEM((32,), jnp.int32),
                   pltpu.VMEM((16,), jnp.int32),
                   pltpu.SemaphoreType.DMA,
                   pltpu.SemaphoreType.DMA],
    compiler_params=pltpu.CompilerParams(needs_layout_passes=False),
    name="edge_bucket",
)


def _segmax_body(C, GC, x_hbm, ssort_hbm, dsort_hbm, cnts_hbm, out_hbm,
                 src_v, dgrp_v, rows_v, acc, cnt_v, sem_a, sem_b):
    NREG = C // 16
    wid = _wid()
    pltpu.sync_copy(cnts_hbm.at[wid], cnt_v)
    cnt = cnt_v[pl.ds(0, 16)][0]

    ninf = jnp.full((16,), -jnp.inf, jnp.float32)

    def init_body(i, c):
        acc[pl.ds(i * 16, 16)] = ninf
        return c
    lax.fori_loop(0, ACC_ROWS * C // 16, init_body, 0)

    nchunks = (cnt + GC - 1) // GC
    sems = (sem_a, sem_b)

    def _lin(g, par):
        pltpu.sync_copy(ssort_hbm.at[wid, pl.ds(g * GC, GC)], src_v.at[par])
        pltpu.sync_copy(dsort_hbm.at[wid, pl.ds(g * GC, GC)], dgrp_v.at[par])

    def _fire(par, sem):
        for j in range(GC // 128):
            pltpu.async_copy(
                x_hbm.at[src_v.at[par].at[pl.ds(j * 128, 128)]],
                rows_v.at[pl.ds(par * GC + j * 128, 128)], sem)

    def _wait(par, sem):
        for j in range(GC // 128):
            pltpu.make_async_copy(
                x_hbm.at[src_v.at[par].at[pl.ds(j * 128, 128)]],
                rows_v.at[pl.ds(par * GC + j * 128, 128)], sem).wait()

    _lin(0, 0)
    _fire(0, sem_a)

    def pair(p, c):
        for par in (0, 1):
            g = 2 * p + par

            @pl.when(g < nchunks)
            def _():
                nxt = g + 1

                @pl.when(nxt < nchunks)
                def _():
                    _lin(nxt, 1 - par)
                    _fire(1 - par, sems[1 - par])

                _wait(par, sems[par])
                ng = jnp.minimum(GC, cnt - g * GC) // 16

                def grp(i, c2):
                    nd = dgrp_v[par, pl.ds(i * 16, 16)][0]
                    rowbase = nd * C
                    for k in range(NREG):
                        vs = [rows_v[par * GC + i * 16 + j, pl.ds(k * 16, 16)]
                              for j in range(16)]
                        while len(vs) > 1:
                            vs = [jnp.maximum(vs[2 * a], vs[2 * a + 1])
                                  for a in range(len(vs) // 2)]
                        a = acc[pl.ds(rowbase + k * 16, 16)]
                        acc[pl.ds(rowbase + k * 16, 16)] = jnp.maximum(a, vs[0])
                    return c2
                lax.fori_loop(0, ng, grp, 0)
        return c

    lax.fori_loop(0, (nchunks + 1) // 2, pair, 0)

    NLAST = N - (NT - 1) * NPT

    @pl.when(wid < NT - 1)
    def _():
        pltpu.sync_copy(acc.at[pl.ds(0, NPT * C)],
                        out_hbm.at[pl.ds(wid * NPT * C, NPT * C)])

    @pl.when(wid == NT - 1)
    def _():
        pltpu.sync_copy(acc.at[pl.ds(0, NLAST * C)],
                        out_hbm.at[pl.ds((NT - 1) * NPT * C, NLAST * C)])


def _make_segmax(C, GC):
    return pl.kernel(
        functools.partial(_segmax_body, C, GC),
        out_type=jax.ShapeDtypeStruct((N * C,), jnp.float32),
        mesh=_MESH,
        scratch_types=[pltpu.VMEM((2, GC), jnp.int32),
                       pltpu.VMEM((2, GC), jnp.int32),
                       pltpu.VMEM((2 * GC, C), jnp.float32),
                       pltpu.VMEM((ACC_ROWS * C,), jnp.float32),
                       pltpu.VMEM((16,), jnp.int32),
                       pltpu.SemaphoreType.DMA,
                       pltpu.SemaphoreType.DMA],
        compiler_params=pltpu.CompilerParams(needs_layout_passes=False,
                                             use_tc_tiling_on_sc=False),
        name=f"segmax{C}",
    )


_segmax128 = _make_segmax(128, 256)
_segmax64 = _make_segmax(64, 768)


def _layer_call(C, residual, x, m, W1, W2, b):
    BN = 1000

    def body(x_ref, m_ref, w1_ref, w2_ref, b_ref, o_ref):
        xb = x_ref[...]
        mb = m_ref[...]
        agg = jnp.where(mb == -jnp.inf, 0.0, mb - xb)
        h = (jnp.dot(xb, w1_ref[...], preferred_element_type=jnp.float32)
             + jnp.dot(agg, w2_ref[...], preferred_element_type=jnp.float32)
             + b_ref[...])
        h = jnp.maximum(h, 0.0)
        if residual:
            h = h + xb
        o_ref[...] = h

    return pl.pallas_call(
        body,
        grid=(N // BN,),
        in_specs=[pl.BlockSpec((BN, C), lambda i: (i, 0)),
                  pl.BlockSpec((BN, C), lambda i: (i, 0)),
                  pl.BlockSpec((C, 64), lambda i: (0, 0)),
                  pl.BlockSpec((C, 64), lambda i: (0, 0)),
                  pl.BlockSpec((1, 64), lambda i: (0, 0))],
        out_specs=pl.BlockSpec((BN, 64), lambda i: (i, 0)),
        out_shape=jax.ShapeDtypeStruct((N, 64), jnp.float32),
    )(x, m, W1, W2, b)


def _tail_call(feats, fusion_W, fusion_b, p1f, p1v, p1b, W2, b2, W3, b3):
    BN = 400

    def body(f_ref, fw_ref, fb_ref, p1f_ref, p1v_ref, p1b_ref,
             w2_ref, b2_ref, w3_ref, b3_ref, o_ref):
        fb = f_ref[...]
        t = jnp.maximum(
            jnp.dot(fb, fw_ref[...], preferred_element_type=jnp.float32)
            + fb_ref[...], 0.0)
        fu = jnp.max(t, axis=1, keepdims=True)
        h1 = jnp.maximum(
            jnp.dot(fb, p1f_ref[...], preferred_element_type=jnp.float32)
            + fu * p1v_ref[...] + p1b_ref[...], 0.0)
        h2 = jnp.maximum(
            jnp.dot(h1, w2_ref[...], preferred_element_type=jnp.float32)
            + b2_ref[...], 0.0)
        o_ref[...] = (jnp.dot(h2, w3_ref[...], preferred_element_type=jnp.float32)
                      + b3_ref[...])

    F = feats.shape[1]
    return pl.pallas_call(
        body,
        grid=(N // BN,),
        in_specs=[pl.BlockSpec((BN, F), lambda i: (i, 0)),
                  pl.BlockSpec((F, 1024), lambda i: (0, 0)),
                  pl.BlockSpec((1, 1024), lambda i: (0, 0)),
                  pl.BlockSpec((F, 512), lambda i: (0, 0)),
                  pl.BlockSpec((1, 512), lambda i: (0, 0)),
                  pl.BlockSpec((1, 512), lambda i: (0, 0)),
                  pl.BlockSpec((512, 256), lambda i: (0, 0)),
                  pl.BlockSpec((1, 256), lambda i: (0, 0)),
                  pl.BlockSpec((256, 13), lambda i: (0, 0)),
                  pl.BlockSpec((1, 13), lambda i: (0, 0))],
        out_specs=pl.BlockSpec((BN, 13), lambda i: (i, 0)),
        out_shape=jax.ShapeDtypeStruct((N, 13), jnp.float32),
    )(feats, fusion_W, fusion_b, p1f, p1v, p1b, W2, b2, W3, b3)


def _pad_ninf(x):
    return jnp.concatenate(
        [x, jnp.full((XROWS - N, x.shape[1]), -jnp.inf, x.dtype)], axis=0)


def kernel(x, edge_index, batch, head_W, head_b, blocks_W, blocks_b,
           fusion_W, fusion_b, pred1_W, pred1_b, pred2_W, pred2_b,
           pred3_W, pred3_b):
    ssort, dsort, cnts = _bucket(edge_index)

    m0 = _segmax128(_pad_ninf(x), ssort, dsort, cnts).reshape(N, 128)
    h = _layer_call(128, False, x, m0, head_W[:128], head_W[128:],
                    head_b.reshape(1, 64))
    feats = [h]
    for i in range(6):
        m = _segmax64(_pad_ninf(h), ssort, dsort, cnts).reshape(N, 64)
        h = _layer_call(64, True, h, m, blocks_W[i, :64], blocks_W[i, 64:],
                        blocks_b[i].reshape(1, 64))
        feats.append(h)
    feats = jnp.concatenate(feats, axis=1)

    return _tail_call(feats, fusion_W, fusion_b.reshape(1, 1024),
                      pred1_W[:448], pred1_W[448:449], pred1_b.reshape(1, 512),
                      pred2_W, pred2_b.reshape(1, 256),
                      pred3_W, pred3_b.reshape(1, 13))
